# baseline (speedup 1.0000x reference)
import functools

import jax
import jax.numpy as jnp
from jax import lax
from jax.experimental import pallas as pl
from jax.experimental.pallas import tpu as pltpu
from jax.experimental.pallas import tpu_sc as plsc

_NC = 2
_NS = 16
_NW = _NC * _NS
_L = 16
_N = 16384
_D = 1024
_RPW = _N // _NW
_G = 16
_DUMP = _RPW + _L
_UNROLL = 8
_KITER = _D // _L // _UNROLL


def _sc_body(x_hbm, p_hbm, m_hbm, sums_hbm, cnts_hbm,
             mask_v, idx_v, xb0, pb0, xb1, pb1, accb, cntb, sem0, sem1):
    wid = lax.axis_index("s") * _NC + lax.axis_index("c")
    base = wid * _RPW

    pltpu.sync_copy(m_hbm.at[pl.ds(base, _RPW)], mask_v)

    fill = jnp.full((_L,), base, jnp.int32)
    for i in range(_RPW // _L + 1):
        idx_v[pl.ds(i * _L, _L)] = fill

    cnt = jnp.int32(0)
    lane = lax.iota(jnp.int32, _L)
    for i in range(_RPW // _L):
        m = mask_v[pl.ds(i * _L, _L)]
        s = m
        for k in (1, 2, 4, 8):
            sh = s.at[jnp.maximum(lane - k, 0)].get(mode="promise_in_bounds")
            s = s + jnp.where(lane >= k, sh, 0)
        pos = jnp.where(m > 0, cnt + s - m, _DUMP + lane)
        ids = (base + i * _L) + lane
        plsc.store_scatter(idx_v, [pos], ids)
        cnt = cnt + s[_L - 1]

    accb[...] = jnp.zeros((_L,), jnp.float32)
    nchunks = (cnt + (_G - 1)) // _G

    def start(ci, xb, pb, sem):
        isl = idx_v.at[pl.ds(ci * _G, _G)]
        pltpu.async_copy(x_hbm.at[isl], xb, sem)
        pltpu.async_copy(p_hbm.at[isl], pb, sem)

    def waitfor(xb, pb, sem):
        pltpu.make_async_copy(x_hbm.at[pl.ds(0, _G)], xb, sem).wait()
        pltpu.make_async_copy(p_hbm.at[pl.ds(0, _G)], pb, sem).wait()

    def compute(ci, xb, pb):
        vrows = jnp.minimum(cnt - ci * _G, _G)

        def row(j, accs):
            def kb(k, accs_):
                a0, a1 = accs_
                for u in range(_UNROLL):
                    off = (k * _UNROLL + u) * _L
                    d = pb[j, pl.ds(off, _L)] - xb[j, pl.ds(off, _L)]
                    if u % 2 == 0:
                        a0 = a0 + d * d
                    else:
                        a1 = a1 + d * d
                return (a0, a1)

            return lax.fori_loop(0, _KITER, kb, accs)

        z = jnp.zeros((_L,), jnp.float32)
        a0, a1 = lax.fori_loop(0, vrows, row, (z, z))
        accb[...] = accb[...] + a0 + a1

    @pl.when(nchunks > 0)
    def _prime():
        start(0, xb0, pb0, sem0)

    def chunk_pair(h, carry):
        c0 = h * 2

        @pl.when(c0 < nchunks)
        def _even():
            @pl.when(c0 + 1 < nchunks)
            def _s1():
                start(c0 + 1, xb1, pb1, sem1)

            waitfor(xb0, pb0, sem0)
            compute(c0, xb0, pb0)

        @pl.when(c0 + 1 < nchunks)
        def _odd():
            @pl.when(c0 + 2 < nchunks)
            def _s0():
                start(c0 + 2, xb0, pb0, sem0)

            waitfor(xb1, pb1, sem1)
            compute(c0 + 1, xb1, pb1)

        return carry

    lax.fori_loop(0, (nchunks + 1) // 2, chunk_pair, jnp.int32(0))

    cntb[...] = jnp.full((_L,), cnt, jnp.int32)
    pltpu.sync_copy(accb, sums_hbm.at[wid])
    pltpu.sync_copy(cntb, cnts_hbm.at[wid])


@jax.jit
def _masked_mse(xf, pf, mi):
    mesh = plsc.VectorSubcoreMesh(core_axis_name="c", subcore_axis_name="s")
    sums, cnts = pl.kernel(
        _sc_body,
        out_type=(
            jax.ShapeDtypeStruct((_NW, _L), jnp.float32),
            jax.ShapeDtypeStruct((_NW, _L), jnp.int32),
        ),
        mesh=mesh,
        compiler_params=pltpu.CompilerParams(needs_layout_passes=False),
        scratch_types=[
            pltpu.VMEM((_RPW,), jnp.int32),
            pltpu.VMEM((_RPW + 2 * _L,), jnp.int32),
            pltpu.VMEM((_G, _D), jnp.float32),
            pltpu.VMEM((_G, _D), jnp.float32),
            pltpu.VMEM((_G, _D), jnp.float32),
            pltpu.VMEM((_G, _D), jnp.float32),
            pltpu.VMEM((_L,), jnp.float32),
            pltpu.VMEM((_L,), jnp.int32),
            pltpu.SemaphoreType.DMA,
            pltpu.SemaphoreType.DMA,
        ],
    )(xf, pf, mi)
    total = jnp.sum(sums)
    cnt = jnp.sum(cnts[:, 0]).astype(jnp.float32) * _D
    loss = total / jnp.maximum(cnt, 1.0)
    return jnp.where(cnt == 0, jnp.asarray(0.0, dtype=xf.dtype), loss)


def kernel(x, predictions, mask):
    b, s, d = x.shape
    n = b * s
    xf = x.reshape(n, d)
    pf = predictions.reshape(n, d)
    mi = mask.reshape(n).astype(jnp.int32)
    return _masked_mse(xf, pf, mi)
